# baseline (speedup 1.0000x reference)
import jax
import jax.numpy as jnp
from jax import lax
from jax.experimental import pallas as pl
from jax.experimental.pallas import tpu as pltpu
from jax.experimental.pallas import tpu_sc as plsc

NC, NS, L = 2, 16, 16
NW = NC * NS

B = 16384
F = 26
BPW = B // NW
IPW = F * BPW


V = 1000000
RS = 15632
RSL = 15520
CH = 4 * RS
FH = 4


def _sc_body(xt_hbm, tablet_hbm, bias_hbm, out_hbm, idx_v, vals_v, bias_v,
             out_v, tab_s, buf0, buf1, sem, sem2, sem3, sem4):
    sid = lax.axis_index("s")
    wid = sid * NC + lax.axis_index("c")
    base = wid * BPW

    table1 = tablet_hbm.at[0]

    def pack_one(f, carry):
        pltpu.async_copy(xt_hbm.at[f, pl.ds(base, BPW)],
                         idx_v.at[pl.ds(f * BPW, BPW)], sem2)
        return carry

    lax.fori_loop(0, F, pack_one, 0)
    pltpu.sync_copy(bias_hbm, bias_v.at[pl.ds(0, 1)])

    def drain_one(f, carry):
        pltpu.make_async_copy(xt_hbm.at[f, pl.ds(base, BPW)],
                              idx_v.at[pl.ds(f * BPW, BPW)], sem2).wait()
        return carry

    lax.fori_loop(0, FH, drain_one, 0)
    pltpu.async_copy(table1.at[idx_v.at[pl.ds(0, FH * BPW)]],
                     vals_v.at[pl.ds(0, FH * BPW)], sem4)

    def stage(off, rs, n):
        bufs = [buf0, buf1]
        for r in range(4):
            b = bufs[r % 2].at[pl.ds(0, rs)]
            if r >= 2:
                pltpu.make_async_copy(
                    b, tab_s.at[pl.ds(off + (r - 2) * rs, rs)], sem3).wait()
            pltpu.sync_copy(table1.at[pl.ds(off + r * rs, rs)], b)
            pltpu.async_copy(b, tab_s.at[pl.ds(off + r * rs, rs)], sem3)
        for r in range(2, 4):
            b = bufs[r % 2].at[pl.ds(0, rs)]
            pltpu.make_async_copy(
                b, tab_s.at[pl.ds(off + r * rs, rs)], sem3).wait()

    @pl.when(sid < 15)
    def _():
        stage(sid * CH, RS, 4)

    @pl.when(sid == 15)
    def _():
        stage(15 * CH, RSL, 4)

    lax.fori_loop(FH, F, drain_one, 0)

    plsc.subcore_barrier()

    pltpu.async_copy(tab_s.at[idx_v.at[pl.ds(FH * BPW, (F - FH) * BPW)]],
                     vals_v.at[pl.ds(FH * BPW, (F - FH) * BPW)], sem)
    pltpu.make_async_copy(table1.at[idx_v.at[pl.ds(0, FH * BPW)]],
                          vals_v.at[pl.ds(0, FH * BPW)], sem4).wait()
    pltpu.make_async_copy(tab_s.at[idx_v.at[pl.ds(FH * BPW, (F - FH) * BPW)]],
                          vals_v.at[pl.ds(FH * BPW, (F - FH) * BPW)],
                          sem).wait()

    bias_vec = jnp.full((L,), bias_v[...][0], jnp.float32)

    def reduce_one(g, carry):
        o = g * L
        acc = bias_vec
        for f in range(F):
            acc = acc + vals_v[pl.ds(f * BPW + o, L)]
        out_v[pl.ds(o, L)] = acc
        return carry

    lax.fori_loop(0, BPW // L, reduce_one, 0)
    pltpu.sync_copy(out_v, out_hbm.at[pl.ds(base, BPW)])


def kernel(X, table, bias):
    Xt = X.astype(jnp.int32).T
    tablet = table.T
    bias1 = bias.astype(jnp.float32)

    mesh = plsc.VectorSubcoreMesh(core_axis_name="c", subcore_axis_name="s")
    out = pl.kernel(
        _sc_body,
        out_type=jax.ShapeDtypeStruct((B,), jnp.float32),
        mesh=mesh,
        scratch_types=[
            pltpu.VMEM((IPW,), jnp.int32),
            pltpu.VMEM((IPW,), jnp.float32),
            pltpu.VMEM((L,), jnp.float32),
            pltpu.VMEM((BPW,), jnp.float32),
            pltpu.VMEM_SHARED((V,), jnp.float32),
            pltpu.VMEM((RS,), jnp.float32),
            pltpu.VMEM((RS,), jnp.float32),
            pltpu.SemaphoreType.DMA,
            pltpu.SemaphoreType.DMA,
            pltpu.SemaphoreType.DMA,
            pltpu.SemaphoreType.DMA,
        ],
    )(Xt, tablet, bias1)
    return out.reshape(B, 1)


# speedup vs baseline: 1.0135x; 1.0135x over previous
import jax
import jax.numpy as jnp
from jax import lax
from jax.experimental import pallas as pl
from jax.experimental.pallas import tpu as pltpu
from jax.experimental.pallas import tpu_sc as plsc

NC, NS, L = 2, 16, 16
NW = NC * NS

B = 16384
F = 26
BPW = B // NW
IPW = F * BPW


V = 1000000
RS = 15632
RSL = 15520
CH = 4 * RS


def _sc_body(xt_hbm, tablet_hbm, bias_hbm, out_hbm, idx_v, vals_v, bias_v,
             out_v, tab_s, buf0, buf1, sem, sem2, sem3):
    sid = lax.axis_index("s")
    wid = sid * NC + lax.axis_index("c")
    base = wid * BPW

    def pack_one(f, carry):
        pltpu.async_copy(xt_hbm.at[f, pl.ds(base, BPW)],
                         idx_v.at[pl.ds(f * BPW, BPW)], sem2)
        return carry

    lax.fori_loop(0, F, pack_one, 0)
    pltpu.sync_copy(bias_hbm, bias_v.at[pl.ds(0, 1)])

    table1 = tablet_hbm.at[0]

    def stage(off, rs, n):
        bufs = [buf0, buf1]
        for r in range(4):
            b = bufs[r % 2].at[pl.ds(0, rs)]
            if r >= 2:
                pltpu.make_async_copy(
                    b, tab_s.at[pl.ds(off + (r - 2) * rs, rs)], sem3).wait()
            pltpu.sync_copy(table1.at[pl.ds(off + r * rs, rs)], b)
            pltpu.async_copy(b, tab_s.at[pl.ds(off + r * rs, rs)], sem3)
        for r in range(2, 4):
            b = bufs[r % 2].at[pl.ds(0, rs)]
            pltpu.make_async_copy(
                b, tab_s.at[pl.ds(off + r * rs, rs)], sem3).wait()

    @pl.when(sid < 15)
    def _():
        stage(sid * CH, RS, 4)

    @pl.when(sid == 15)
    def _():
        stage(15 * CH, RSL, 4)

    def drain_one(f, carry):
        pltpu.make_async_copy(xt_hbm.at[f, pl.ds(base, BPW)],
                              idx_v.at[pl.ds(f * BPW, BPW)], sem2).wait()
        return carry

    lax.fori_loop(0, F, drain_one, 0)

    plsc.subcore_barrier()

    pltpu.async_copy(tab_s.at[idx_v], vals_v, sem).wait()

    bias_vec = jnp.full((L,), bias_v[...][0], jnp.float32)

    def reduce_one(g, carry):
        o = g * L
        acc = bias_vec
        for f in range(F):
            acc = acc + vals_v[pl.ds(f * BPW + o, L)]
        out_v[pl.ds(o, L)] = acc
        return carry

    lax.fori_loop(0, BPW // L, reduce_one, 0)
    pltpu.sync_copy(out_v, out_hbm.at[pl.ds(base, BPW)])


def kernel(X, table, bias):
    Xt = X.astype(jnp.int32).T
    tablet = table.T
    bias1 = bias.astype(jnp.float32)

    mesh = plsc.VectorSubcoreMesh(core_axis_name="c", subcore_axis_name="s")
    out = pl.kernel(
        _sc_body,
        out_type=jax.ShapeDtypeStruct((B,), jnp.float32),
        mesh=mesh,
        scratch_types=[
            pltpu.VMEM((IPW,), jnp.int32),
            pltpu.VMEM((IPW,), jnp.float32),
            pltpu.VMEM((L,), jnp.float32),
            pltpu.VMEM((BPW,), jnp.float32),
            pltpu.VMEM_SHARED((V,), jnp.float32),
            pltpu.VMEM((RS,), jnp.float32),
            pltpu.VMEM((RS,), jnp.float32),
            pltpu.SemaphoreType.DMA,
            pltpu.SemaphoreType.DMA,
            pltpu.SemaphoreType.DMA,
        ],
    )(Xt, tablet, bias1)
    return out.reshape(B, 1)
